# baseline (speedup 1.0000x reference)
import functools

import jax
import jax.numpy as jnp
from jax import lax
from jax.experimental import pallas as pl
from jax.experimental.pallas import tpu as pltpu
from jax.experimental.pallas import tpu_sc as plsc

NC = 2
NS = 16
L = 16


def _make_transpose2(B, D, idx_dtype, src_dtype):
    RBT = 2048

    def body(i_ref, s_ref, it_ref, st_ref):
        it_ref[...] = i_ref[...].T
        st_ref[...] = s_ref[...].T

    return pl.pallas_call(
        body,
        grid=(pl.cdiv(B, RBT),),
        in_specs=[
            pl.BlockSpec((RBT, D), lambda i: (i, 0)),
            pl.BlockSpec((RBT, D), lambda i: (i, 0)),
        ],
        out_specs=[
            pl.BlockSpec((D, RBT), lambda i: (0, i)),
            pl.BlockSpec((D, RBT), lambda i: (0, i)),
        ],
        out_shape=[
            jax.ShapeDtypeStruct((D, B), idx_dtype),
            jax.ShapeDtypeStruct((D, B), src_dtype),
        ],
    )


def _make_combine(M, D):
    RM = 1024

    def body(x_ref, dt_ref, o_ref):
        o_ref[...] = x_ref[...] + dt_ref[...].T

    return pl.pallas_call(
        body,
        grid=(pl.cdiv(M, RM),),
        in_specs=[
            pl.BlockSpec((RM, D), lambda i: (i, 0)),
            pl.BlockSpec((D, RM), lambda i: (0, i)),
        ],
        out_specs=pl.BlockSpec((RM, D), lambda i: (i, 0)),
        out_shape=jax.ShapeDtypeStruct((M, D), jnp.float32),
    )


def _make_sc_kernel(M, D, B):
    total = M * D
    NCHUNK = 4
    assert D == NCHUNK * NS
    CW = NS * M
    NPASS = NCHUNK // NC
    PW = CW // NS
    assert PW % 8 == 0
    BLK = 2048
    assert B % BLK == 0
    NBLK = B // BLK
    NVEC = BLK // L
    SW = 10000
    assert PW % SW == 0 and SW % 8 == 0
    NSTAGE = PW // SW

    mesh = plsc.VectorSubcoreMesh(core_axis_name="c", subcore_axis_name="s")

    @functools.partial(
        pl.kernel,
        mesh=mesh,
        out_type=jax.ShapeDtypeStruct((total,), jnp.float32),
        scratch_types=[
            pltpu.VMEM_SHARED((CW + 16,), jnp.float32),
            pltpu.VMEM((BLK,), jnp.int32),
            pltpu.VMEM((BLK,), jnp.int32),
            pltpu.VMEM((BLK,), jnp.float32),
            pltpu.VMEM((BLK,), jnp.float32),
            pltpu.VMEM((BLK,), jnp.int32),
            pltpu.VMEM((BLK,), jnp.int32),
            pltpu.VMEM((SW,), jnp.float32),
            pltpu.SemaphoreType.DMA,
        ],
    )
    def scatter_add_kernel(idxt_hbm, srct_hbm, outt_hbm,
                           accum, idx_raw0, idx_raw1, src_buf0, src_buf1,
                           idx_scat0, idx_scat1, stage, sem):
        idx_raw = (idx_raw0, idx_raw1)
        src_buf = (src_buf0, src_buf1)
        idx_scat = (idx_scat0, idx_scat1)
        c = lax.axis_index("c")
        s = lax.axis_index("s")
        zeros16 = jnp.zeros((L,), jnp.float32)

        for p in range(NPASS):
            k = p * NC + c
            base = k * CW
            colbase = (k * NS + s) * B

            def zero_body(i, _):
                stage[pl.ds(i * L, L)] = zeros16
                return 0

            lax.fori_loop(0, SW // L, zero_body, 0)

            def init_body(t, _):
                pltpu.sync_copy(stage, accum.at[pl.ds(s * PW + t * SW, SW)])
                return 0

            lax.fori_loop(0, NSTAGE, init_body, 0)
            plsc.subcore_barrier()

            for b in range(NBLK):
                d = b % 2
                pltpu.sync_copy(idxt_hbm.at[pl.ds(colbase + b * BLK, BLK)],
                                idx_raw[d])
                pltpu.sync_copy(srct_hbm.at[pl.ds(colbase + b * BLK, BLK)],
                                src_buf[d])

                def vec_body(i, _, d=d):
                    v = idx_raw[d][pl.ds(i * L, L)]
                    idx_scat[d][pl.ds(i * L, L)] = v + s * M
                    return 0

                lax.fori_loop(0, NVEC, vec_body, 0)
                if b >= 1:
                    pltpu.make_async_copy(src_buf[1 - d],
                                          accum.at[idx_scat[1 - d]],
                                          sem).wait()
                pltpu.async_copy(src_buf[d], accum.at[idx_scat[d]],
                                 sem, add=True)
            pltpu.make_async_copy(src_buf[(NBLK - 1) % 2],
                                  accum.at[idx_scat[(NBLK - 1) % 2]],
                                  sem).wait()
            plsc.subcore_barrier()

            def wb_body(t, _):
                pltpu.sync_copy(accum.at[pl.ds(s * PW + t * SW, SW)], stage)
                pltpu.sync_copy(stage,
                                outt_hbm.at[pl.ds(base + s * PW + t * SW, SW)])
                return 0

            lax.fori_loop(0, NSTAGE, wb_body, 0)
            plsc.subcore_barrier()

    return scatter_add_kernel


def kernel(x, dim, index, src, out):
    M, D = x.shape
    B = src.shape[0]
    del out
    rows = index + jnp.asarray(dim, dtype=index.dtype)
    idxt, srct = _make_transpose2(B, D, rows.dtype, src.dtype)(rows, src)
    deltat = _make_sc_kernel(M, D, B)(idxt.reshape(-1), srct.reshape(-1))
    return _make_combine(M, D)(x, deltat.reshape(D, M))


# speedup vs baseline: 1.8244x; 1.8244x over previous
import functools

import jax
import jax.numpy as jnp
from jax import lax
from jax.experimental import pallas as pl
from jax.experimental.pallas import tpu as pltpu
from jax.experimental.pallas import tpu_sc as plsc

NC = 2
NS = 16
L = 16


def _make_sc_kernel(M, D, B):
    total = M * D
    NCHUNK = 4
    assert D == NCHUNK * NS
    CW = NS * M
    NPASS = NCHUNK // NC
    PW = CW // NS
    assert PW % 8 == 0
    BLK = 4096
    assert B % BLK == 0
    NBLK = B // BLK
    NVEC = BLK // L

    mesh = plsc.VectorSubcoreMesh(core_axis_name="c", subcore_axis_name="s")

    @functools.partial(
        pl.kernel,
        mesh=mesh,
        out_type=jax.ShapeDtypeStruct((total,), jnp.float32),
        compiler_params=pltpu.CompilerParams(use_tc_tiling_on_sc=False),
        scratch_types=[
            pltpu.VMEM_SHARED((CW + 16,), jnp.float32),
            pltpu.VMEM((BLK,), jnp.int32),
            pltpu.VMEM((BLK,), jnp.int32),
            pltpu.VMEM((BLK,), jnp.float32),
            pltpu.VMEM((BLK,), jnp.float32),
            pltpu.VMEM((BLK,), jnp.int32),
            pltpu.VMEM((BLK,), jnp.int32),
            pltpu.SemaphoreType.DMA,
            pltpu.SemaphoreType.DMA,
        ],
    )
    def scatter_add_kernel(xt_hbm, idxt_hbm, srct_hbm, outt_hbm,
                           accum, idx_raw0, idx_raw1, src_buf0, src_buf1,
                           idx_scat0, idx_scat1, sem, lsem):
        idx_raw = (idx_raw0, idx_raw1)
        src_buf = (src_buf0, src_buf1)
        idx_scat = (idx_scat0, idx_scat1)
        c = lax.axis_index("c")
        s = lax.axis_index("s")

        for p in range(NPASS):
            k = p * NC + c
            base = k * CW
            colbase = (k * NS + s) * B

            pltpu.sync_copy(xt_hbm.at[pl.ds(base + s * PW, PW)],
                            accum.at[pl.ds(s * PW, PW)])
            plsc.subcore_barrier()

            def islice(b):
                return idxt_hbm.at[pl.ds(colbase + b * BLK, BLK)]

            def sslice(b):
                return srct_hbm.at[pl.ds(colbase + b * BLK, BLK)]

            pltpu.async_copy(islice(0), idx_raw[0], lsem)
            pltpu.async_copy(sslice(0), src_buf[0], lsem)
            for b in range(NBLK):
                d = b % 2
                pltpu.make_async_copy(islice(b), idx_raw[d], lsem).wait()
                pltpu.make_async_copy(sslice(b), src_buf[d], lsem).wait()

                def vec_body(i, _, d=d):
                    v = idx_raw[d][pl.ds(i * L, L)]
                    idx_scat[d][pl.ds(i * L, L)] = v + s * M
                    return 0

                lax.fori_loop(0, NVEC, vec_body, 0)
                if b >= 1:
                    pltpu.make_async_copy(src_buf[1 - d],
                                          accum.at[idx_scat[1 - d]],
                                          sem).wait()
                if b + 1 < NBLK:
                    pltpu.async_copy(islice(b + 1), idx_raw[1 - d], lsem)
                    pltpu.async_copy(sslice(b + 1), src_buf[1 - d], lsem)
                pltpu.async_copy(src_buf[d], accum.at[idx_scat[d]],
                                 sem, add=True)
            pltpu.make_async_copy(src_buf[(NBLK - 1) % 2],
                                  accum.at[idx_scat[(NBLK - 1) % 2]],
                                  sem).wait()
            plsc.subcore_barrier()

            pltpu.sync_copy(accum.at[pl.ds(s * PW, PW)],
                            outt_hbm.at[pl.ds(base + s * PW, PW)])
            plsc.subcore_barrier()

    return scatter_add_kernel


def kernel(x, dim, index, src, out):
    M, D = x.shape
    B = src.shape[0]
    del out
    rows = index + jnp.asarray(dim, dtype=index.dtype)
    sc = _make_sc_kernel(M, D, B)
    outt = sc(x.T.reshape(-1), rows.T.reshape(-1), src.T.reshape(-1))
    return outt.reshape(D, M).T
